# baseline (device time: 20742 ns/iter reference)
import jax
import jax.numpy as jnp
from jax import lax
from jax.experimental import pallas as pl
from jax.experimental.pallas import tpu as pltpu

N_LAYERS = 3
C = 4


def kernel(x, Win0, Wout0, Win1, Wout1, Win2, Wout2):
    b, dy = x.shape
    dk, hx = Win0.shape
    hb = b // C

    def body(x_ref, win0_ref, wout0_ref, win1_ref, wout1_ref, win2_ref,
             wout2_ref, out_ref,
             h_send, h_recv, q_send, q_recv, p_send, p_recv,
             win_send, win_recv, wout_send, wout_recv,
             y0_ssem, y0_rsem, x0_ssem, x0_rsem,
             p_ssem, p_rsem, win_ssem, win_rsem, wout_ssem, wout_rsem):
        mx = lax.axis_index("x")
        my = lax.axis_index("y")
        y_partner = (mx, 1 - my)
        x_partner = (1 - mx, my)

        barrier = pltpu.get_barrier_semaphore()
        for nbr in (y_partner, x_partner):
            pl.semaphore_signal(
                barrier, inc=1, device_id=nbr,
                device_id_type=pl.DeviceIdType.MESH,
            )
        pl.semaphore_wait(barrier, 2)

        wins = [win0_ref, win1_ref, win2_ref]
        wouts = [wout0_ref, wout1_ref, wout2_ref]
        inflight = []

        def start_rdma(src, dst, ssem, rsem, partner):
            rdma = pltpu.make_async_remote_copy(
                src_ref=src, dst_ref=dst, send_sem=ssem, recv_sem=rsem,
                device_id=partner, device_id_type=pl.DeviceIdType.MESH,
            )
            rdma.start()
            inflight.append(rdma)
            return rdma

        def send_weights(l):
            win_send[l] = wins[l][...].astype(jnp.bfloat16)
            w = start_rdma(
                win_send.at[l], win_recv.at[l],
                win_ssem.at[l], win_rsem.at[l], x_partner,
            )
            wout_send[l] = wouts[l][...].astype(jnp.bfloat16)
            wo = start_rdma(
                wout_send.at[l], wout_recv.at[l],
                wout_ssem.at[l], wout_rsem.at[l], x_partner,
            )
            return w, wo

        waited = set()

        def wait_once(key, rdma):
            if key not in waited:
                rdma.wait_recv()
                waited.add(key)

        w1, wo1 = send_weights(1)

        curs = [
            x_ref[c * hb:(c + 1) * hb, :].astype(jnp.bfloat16)
            for c in range(C)
        ]
        win0b = win0_ref[...].astype(jnp.bfloat16)
        p0 = [None] * C
        ry = [None] * C
        for c in range(C):
            p0[c] = jnp.dot(
                curs[c], win0b, preferred_element_type=jnp.float32
            ).astype(jnp.bfloat16)
            h_send[c] = p0[c]
            ry[c] = start_rdma(
                h_send.at[c], h_recv.at[c], y0_ssem.at[c], y0_rsem.at[c],
                y_partner,
            )
        wout0b = wout0_ref[...].astype(jnp.bfloat16)
        q0 = [None] * C
        rx = [None] * C
        for c in range(C):
            ry[c].wait_recv()
            h0 = jnp.maximum(p0[c] + h_recv[c], 0.0)
            q0[c] = jnp.dot(
                h0, wout0b, preferred_element_type=jnp.float32
            ).astype(jnp.bfloat16)
            q_send[c] = q0[c]
            rx[c] = start_rdma(
                q_send.at[c], q_recv.at[c], x0_ssem.at[c], x0_rsem.at[c],
                x_partner,
            )
        w2, wo2 = send_weights(2)

        def make_l0_finalize(c):
            def fin():
                rx[c].wait_recv()
                return q0[c] + q_recv[c]
            return fin

        pending = [make_l0_finalize(c) for c in range(C)]

        w_rdmas = {1: (w1, wo1), 2: (w2, wo2)}
        qs = [None] * C
        for l in (1, 2):
            win_rd, wout_rd = w_rdmas[l]
            winb = wins[l][...].astype(jnp.bfloat16)
            p1 = [None] * C
            p2 = [None] * C
            r0 = [None] * C
            r1 = [None] * C
            for c in range(C):
                curs[c] = pending[c]()
                p1[c] = jnp.dot(
                    curs[c], winb, preferred_element_type=jnp.float32
                ).astype(jnp.bfloat16)
                p_send[l, 0, c] = p1[c]
                r0[c] = start_rdma(
                    p_send.at[l, 0, c], p_recv.at[l, 0, c],
                    p_ssem.at[l, 0, c], p_rsem.at[l, 0, c], y_partner,
                )
            wait_once(("win", l), win_rd)
            for c in range(C):
                p2[c] = jnp.dot(
                    curs[c], win_recv[l], preferred_element_type=jnp.float32
                ).astype(jnp.bfloat16)
                p_send[l, 1, c] = p2[c]
                r1[c] = start_rdma(
                    p_send.at[l, 1, c], p_recv.at[l, 1, c],
                    p_ssem.at[l, 1, c], p_rsem.at[l, 1, c], y_partner,
                )
            woutb = wouts[l][...].astype(jnp.bfloat16)
            for c in range(C):
                r0[c].wait_recv()
                h1 = jnp.maximum(p1[c] + p_recv[l, 0, c], 0.0)
                qs[c] = jnp.dot(h1, woutb, preferred_element_type=jnp.float32)

            def make_finalize(l, c, wout_rd=wout_rd, r1c=None, p2c=None):
                def fin():
                    wait_once(("wout", l), wout_rd)
                    r1c.wait_recv()
                    h2 = jnp.maximum(p2c + p_recv[l, 1, c], 0.0)
                    q = qs[c] + jnp.dot(
                        h2, wout_recv[l], preferred_element_type=jnp.float32
                    )
                    qs[c] = q
                    return q.astype(jnp.bfloat16)
                return fin

            pending = [
                make_finalize(l, c, r1c=r1[c], p2c=p2[c]) for c in range(C)
            ]

        for c in range(C):
            pending[c]()
            out_ref[c * hb:(c + 1) * hb, :] = qs[c]
        for rdma in inflight:
            rdma.wait_send()

    return pl.pallas_call(
        body,
        out_shape=jax.ShapeDtypeStruct((b, dy), jnp.float32),
        in_specs=[pl.BlockSpec(memory_space=pltpu.VMEM)] * 7,
        out_specs=pl.BlockSpec(memory_space=pltpu.VMEM),
        scratch_shapes=[
            pltpu.VMEM((C, hb, hx), jnp.bfloat16),
            pltpu.VMEM((C, hb, hx), jnp.bfloat16),
            pltpu.VMEM((C, hb, dy), jnp.bfloat16),
            pltpu.VMEM((C, hb, dy), jnp.bfloat16),
            pltpu.VMEM((N_LAYERS, 2, C, hb, hx), jnp.bfloat16),
            pltpu.VMEM((N_LAYERS, 2, C, hb, hx), jnp.bfloat16),
            pltpu.VMEM((N_LAYERS, dk, hx), jnp.bfloat16),
            pltpu.VMEM((N_LAYERS, dk, hx), jnp.bfloat16),
            pltpu.VMEM((N_LAYERS, hx, dy), jnp.bfloat16),
            pltpu.VMEM((N_LAYERS, hx, dy), jnp.bfloat16),
            pltpu.SemaphoreType.DMA((C,)),
            pltpu.SemaphoreType.DMA((C,)),
            pltpu.SemaphoreType.DMA((C,)),
            pltpu.SemaphoreType.DMA((C,)),
            pltpu.SemaphoreType.DMA((N_LAYERS, 2, C)),
            pltpu.SemaphoreType.DMA((N_LAYERS, 2, C)),
            pltpu.SemaphoreType.DMA((N_LAYERS,)),
            pltpu.SemaphoreType.DMA((N_LAYERS,)),
            pltpu.SemaphoreType.DMA((N_LAYERS,)),
            pltpu.SemaphoreType.DMA((N_LAYERS,)),
        ],
        compiler_params=pltpu.CompilerParams(collective_id=0),
    )(x, Win0, Wout0, Win1, Wout1, Win2, Wout2)


# device time: 20634 ns/iter; 1.0052x vs baseline; 1.0052x over previous
import jax
import jax.numpy as jnp
from jax import lax
from jax.experimental import pallas as pl
from jax.experimental.pallas import tpu as pltpu

N_LAYERS = 3
C = 2


def kernel(x, Win0, Wout0, Win1, Wout1, Win2, Wout2):
    b, dy = x.shape
    dk, hx = Win0.shape
    hb = b // C

    def body(x_ref, win0_ref, wout0_ref, win1_ref, wout1_ref, win2_ref,
             wout2_ref, out_ref,
             h_send, h_recv, q_send, q_recv, p_send, p_recv,
             win_send, win_recv, wout_send, wout_recv,
             y0_ssem, y0_rsem, x0_ssem, x0_rsem,
             p_ssem, p_rsem, win_ssem, win_rsem, wout_ssem, wout_rsem):
        mx = lax.axis_index("x")
        my = lax.axis_index("y")
        y_partner = (mx, 1 - my)
        x_partner = (1 - mx, my)

        wins = [win0_ref, win1_ref, win2_ref]
        wouts = [wout0_ref, wout1_ref, wout2_ref]
        inflight = []

        def start_rdma(src, dst, ssem, rsem, partner):
            rdma = pltpu.make_async_remote_copy(
                src_ref=src, dst_ref=dst, send_sem=ssem, recv_sem=rsem,
                device_id=partner, device_id_type=pl.DeviceIdType.MESH,
            )
            rdma.start()
            inflight.append(rdma)
            return rdma

        def send_weights(l):
            win_send[l] = wins[l][...].astype(jnp.bfloat16)
            w = start_rdma(
                win_send.at[l], win_recv.at[l],
                win_ssem.at[l], win_rsem.at[l], x_partner,
            )
            wout_send[l] = wouts[l][...].astype(jnp.bfloat16)
            wo = start_rdma(
                wout_send.at[l], wout_recv.at[l],
                wout_ssem.at[l], wout_rsem.at[l], x_partner,
            )
            return w, wo

        waited = set()

        def wait_once(key, rdma):
            if key not in waited:
                rdma.wait_recv()
                waited.add(key)

        curs = [
            x_ref[c * hb:(c + 1) * hb, :].astype(jnp.bfloat16)
            for c in range(C)
        ]
        win0b = win0_ref[...].astype(jnp.bfloat16)
        p0 = [None] * C
        ry = [None] * C
        for c in range(C):
            p0[c] = jnp.dot(
                curs[c], win0b, preferred_element_type=jnp.float32
            ).astype(jnp.bfloat16)
            h_send[c] = p0[c]
        win_send[1] = win1_ref[...].astype(jnp.bfloat16)
        wout_send[1] = wout1_ref[...].astype(jnp.bfloat16)

        barrier = pltpu.get_barrier_semaphore()
        for nbr in (y_partner, x_partner):
            pl.semaphore_signal(
                barrier, inc=1, device_id=nbr,
                device_id_type=pl.DeviceIdType.MESH,
            )
        pl.semaphore_wait(barrier, 2)

        for c in range(C):
            ry[c] = start_rdma(
                h_send.at[c], h_recv.at[c], y0_ssem.at[c], y0_rsem.at[c],
                y_partner,
            )
        w1 = start_rdma(
            win_send.at[1], win_recv.at[1],
            win_ssem.at[1], win_rsem.at[1], x_partner,
        )
        wo1 = start_rdma(
            wout_send.at[1], wout_recv.at[1],
            wout_ssem.at[1], wout_rsem.at[1], x_partner,
        )
        wout0b = wout0_ref[...].astype(jnp.bfloat16)
        q0 = [None] * C
        rx = [None] * C
        for c in range(C):
            ry[c].wait_recv()
            h0 = jnp.maximum(p0[c] + h_recv[c], 0.0)
            q0[c] = jnp.dot(
                h0, wout0b, preferred_element_type=jnp.float32
            ).astype(jnp.bfloat16)
            q_send[c] = q0[c]
            rx[c] = start_rdma(
                q_send.at[c], q_recv.at[c], x0_ssem.at[c], x0_rsem.at[c],
                x_partner,
            )
        w2, wo2 = send_weights(2)

        def make_l0_finalize(c):
            def fin():
                rx[c].wait_recv()
                return q0[c] + q_recv[c]
            return fin

        pending = [make_l0_finalize(c) for c in range(C)]

        w_rdmas = {1: (w1, wo1), 2: (w2, wo2)}
        qs = [None] * C
        for l in (1, 2):
            win_rd, wout_rd = w_rdmas[l]
            winb = wins[l][...].astype(jnp.bfloat16)
            p1 = [None] * C
            p2 = [None] * C
            r0 = [None] * C
            r1 = [None] * C
            for c in range(C):
                curs[c] = pending[c]()
                p1[c] = jnp.dot(
                    curs[c], winb, preferred_element_type=jnp.float32
                ).astype(jnp.bfloat16)
                p_send[l, 0, c] = p1[c]
                r0[c] = start_rdma(
                    p_send.at[l, 0, c], p_recv.at[l, 0, c],
                    p_ssem.at[l, 0, c], p_rsem.at[l, 0, c], y_partner,
                )
            wait_once(("win", l), win_rd)
            for c in range(C):
                p2[c] = jnp.dot(
                    curs[c], win_recv[l], preferred_element_type=jnp.float32
                ).astype(jnp.bfloat16)
                p_send[l, 1, c] = p2[c]
                r1[c] = start_rdma(
                    p_send.at[l, 1, c], p_recv.at[l, 1, c],
                    p_ssem.at[l, 1, c], p_rsem.at[l, 1, c], y_partner,
                )
            woutb = wouts[l][...].astype(jnp.bfloat16)
            for c in range(C):
                r0[c].wait_recv()
                h1 = jnp.maximum(p1[c] + p_recv[l, 0, c], 0.0)
                qs[c] = jnp.dot(h1, woutb, preferred_element_type=jnp.float32)

            def make_finalize(l, c, wout_rd=wout_rd, r1c=None, p2c=None):
                def fin():
                    wait_once(("wout", l), wout_rd)
                    r1c.wait_recv()
                    h2 = jnp.maximum(p2c + p_recv[l, 1, c], 0.0)
                    q = qs[c] + jnp.dot(
                        h2, wout_recv[l], preferred_element_type=jnp.float32
                    )
                    qs[c] = q
                    return q.astype(jnp.bfloat16)
                return fin

            pending = [
                make_finalize(l, c, r1c=r1[c], p2c=p2[c]) for c in range(C)
            ]

        for c in range(C):
            pending[c]()
            out_ref[c * hb:(c + 1) * hb, :] = qs[c]
        for rdma in inflight:
            rdma.wait_send()

    return pl.pallas_call(
        body,
        out_shape=jax.ShapeDtypeStruct((b, dy), jnp.float32),
        in_specs=[pl.BlockSpec(memory_space=pltpu.VMEM)] * 7,
        out_specs=pl.BlockSpec(memory_space=pltpu.VMEM),
        scratch_shapes=[
            pltpu.VMEM((C, hb, hx), jnp.bfloat16),
            pltpu.VMEM((C, hb, hx), jnp.bfloat16),
            pltpu.VMEM((C, hb, dy), jnp.bfloat16),
            pltpu.VMEM((C, hb, dy), jnp.bfloat16),
            pltpu.VMEM((N_LAYERS, 2, C, hb, hx), jnp.bfloat16),
            pltpu.VMEM((N_LAYERS, 2, C, hb, hx), jnp.bfloat16),
            pltpu.VMEM((N_LAYERS, dk, hx), jnp.bfloat16),
            pltpu.VMEM((N_LAYERS, dk, hx), jnp.bfloat16),
            pltpu.VMEM((N_LAYERS, hx, dy), jnp.bfloat16),
            pltpu.VMEM((N_LAYERS, hx, dy), jnp.bfloat16),
            pltpu.SemaphoreType.DMA((C,)),
            pltpu.SemaphoreType.DMA((C,)),
            pltpu.SemaphoreType.DMA((C,)),
            pltpu.SemaphoreType.DMA((C,)),
            pltpu.SemaphoreType.DMA((N_LAYERS, 2, C)),
            pltpu.SemaphoreType.DMA((N_LAYERS, 2, C)),
            pltpu.SemaphoreType.DMA((N_LAYERS,)),
            pltpu.SemaphoreType.DMA((N_LAYERS,)),
            pltpu.SemaphoreType.DMA((N_LAYERS,)),
            pltpu.SemaphoreType.DMA((N_LAYERS,)),
        ],
        compiler_params=pltpu.CompilerParams(collective_id=0),
    )(x, Win0, Wout0, Win1, Wout1, Win2, Wout2)


# device time: 20633 ns/iter; 1.0053x vs baseline; 1.0000x over previous
import jax
import jax.numpy as jnp
from jax import lax
from jax.experimental import pallas as pl
from jax.experimental.pallas import tpu as pltpu

N_LAYERS = 3
C = 2


def kernel(x, Win0, Wout0, Win1, Wout1, Win2, Wout2):
    b, dy = x.shape
    dk, hx = Win0.shape
    hb = b // C

    def body(x_ref, win0_ref, wout0_ref, win1_ref, wout1_ref, win2_ref,
             wout2_ref, out_ref,
             h_send, h_recv, q_send, q_recv, p_send, p_recv,
             win_send, win_recv, wout_send, wout_recv,
             y0_ssem, y0_rsem, x0_ssem, x0_rsem,
             p_ssem, p_rsem, win_ssem, win_rsem, wout_ssem, wout_rsem):
        mx = lax.axis_index("x")
        my = lax.axis_index("y")
        y_partner = (mx, 1 - my)
        x_partner = (1 - mx, my)

        wins = [win0_ref, win1_ref, win2_ref]
        wouts = [wout0_ref, wout1_ref, wout2_ref]
        inflight = []

        def start_rdma(src, dst, ssem, rsem, partner):
            rdma = pltpu.make_async_remote_copy(
                src_ref=src, dst_ref=dst, send_sem=ssem, recv_sem=rsem,
                device_id=partner, device_id_type=pl.DeviceIdType.MESH,
            )
            rdma.start()
            inflight.append(rdma)
            return rdma

        def send_weights(l):
            win_send[l] = wins[l][...].astype(jnp.bfloat16)
            w = start_rdma(
                win_send.at[l], win_recv.at[l],
                win_ssem.at[l], win_rsem.at[l], x_partner,
            )
            wout_send[l] = wouts[l][...].astype(jnp.bfloat16)
            wo = start_rdma(
                wout_send.at[l], wout_recv.at[l],
                wout_ssem.at[l], wout_rsem.at[l], x_partner,
            )
            return w, wo

        waited = set()

        def wait_once(key, rdma):
            if key not in waited:
                rdma.wait_recv()
                waited.add(key)

        curs = [
            x_ref[c * hb:(c + 1) * hb, :].astype(jnp.bfloat16)
            for c in range(C)
        ]
        win0b = win0_ref[...].astype(jnp.bfloat16)
        p0 = [None] * C
        ry = [None] * C
        for c in range(C):
            p0[c] = jnp.dot(
                curs[c], win0b, preferred_element_type=jnp.float32
            ).astype(jnp.bfloat16)
            h_send[c] = p0[c]
        win_send[1] = win1_ref[...].astype(jnp.bfloat16)
        wout_send[1] = wout1_ref[...].astype(jnp.bfloat16)

        barrier = pltpu.get_barrier_semaphore()
        for nbr in (y_partner, x_partner):
            pl.semaphore_signal(
                barrier, inc=1, device_id=nbr,
                device_id_type=pl.DeviceIdType.MESH,
            )
        pl.semaphore_wait(barrier, 2)

        for c in range(C):
            ry[c] = start_rdma(
                h_send.at[c], h_recv.at[c], y0_ssem.at[c], y0_rsem.at[c],
                y_partner,
            )
        w1 = start_rdma(
            win_send.at[1], win_recv.at[1],
            win_ssem.at[1], win_rsem.at[1], x_partner,
        )
        wout0b = wout0_ref[...].astype(jnp.bfloat16)
        q0 = [None] * C
        rx = [None] * C
        for c in range(C):
            ry[c].wait_recv()
            h0 = jnp.maximum(p0[c] + h_recv[c], 0.0)
            q0[c] = jnp.dot(
                h0, wout0b, preferred_element_type=jnp.float32
            ).astype(jnp.bfloat16)
            q_send[c] = q0[c]
            rx[c] = start_rdma(
                q_send.at[c], q_recv.at[c], x0_ssem.at[c], x0_rsem.at[c],
                x_partner,
            )
        wo1 = start_rdma(
            wout_send.at[1], wout_recv.at[1],
            wout_ssem.at[1], wout_rsem.at[1], x_partner,
        )
        w2, wo2 = send_weights(2)

        def make_l0_finalize(c):
            def fin():
                rx[c].wait_recv()
                return q0[c] + q_recv[c]
            return fin

        pending = [make_l0_finalize(c) for c in range(C)]

        w_rdmas = {1: (w1, wo1), 2: (w2, wo2)}
        qs = [None] * C
        for l in (1, 2):
            win_rd, wout_rd = w_rdmas[l]
            winb = wins[l][...].astype(jnp.bfloat16)
            p1 = [None] * C
            p2 = [None] * C
            r0 = [None] * C
            r1 = [None] * C
            for c in range(C):
                curs[c] = pending[c]()
                p1[c] = jnp.dot(
                    curs[c], winb, preferred_element_type=jnp.float32
                ).astype(jnp.bfloat16)
                p_send[l, 0, c] = p1[c]
                r0[c] = start_rdma(
                    p_send.at[l, 0, c], p_recv.at[l, 0, c],
                    p_ssem.at[l, 0, c], p_rsem.at[l, 0, c], y_partner,
                )
            wait_once(("win", l), win_rd)
            for c in range(C):
                p2[c] = jnp.dot(
                    curs[c], win_recv[l], preferred_element_type=jnp.float32
                ).astype(jnp.bfloat16)
                p_send[l, 1, c] = p2[c]
                r1[c] = start_rdma(
                    p_send.at[l, 1, c], p_recv.at[l, 1, c],
                    p_ssem.at[l, 1, c], p_rsem.at[l, 1, c], y_partner,
                )
            woutb = wouts[l][...].astype(jnp.bfloat16)
            for c in range(C):
                r0[c].wait_recv()
                h1 = jnp.maximum(p1[c] + p_recv[l, 0, c], 0.0)
                qs[c] = jnp.dot(h1, woutb, preferred_element_type=jnp.float32)

            def make_finalize(l, c, wout_rd=wout_rd, r1c=None, p2c=None):
                def fin():
                    wait_once(("wout", l), wout_rd)
                    r1c.wait_recv()
                    h2 = jnp.maximum(p2c + p_recv[l, 1, c], 0.0)
                    q = qs[c] + jnp.dot(
                        h2, wout_recv[l], preferred_element_type=jnp.float32
                    )
                    qs[c] = q
                    return q.astype(jnp.bfloat16)
                return fin

            pending = [
                make_finalize(l, c, r1c=r1[c], p2c=p2[c]) for c in range(C)
            ]

        for c in range(C):
            pending[c]()
            out_ref[c * hb:(c + 1) * hb, :] = qs[c]
        for rdma in inflight:
            rdma.wait_send()

    return pl.pallas_call(
        body,
        out_shape=jax.ShapeDtypeStruct((b, dy), jnp.float32),
        in_specs=[pl.BlockSpec(memory_space=pltpu.VMEM)] * 7,
        out_specs=pl.BlockSpec(memory_space=pltpu.VMEM),
        scratch_shapes=[
            pltpu.VMEM((C, hb, hx), jnp.bfloat16),
            pltpu.VMEM((C, hb, hx), jnp.bfloat16),
            pltpu.VMEM((C, hb, dy), jnp.bfloat16),
            pltpu.VMEM((C, hb, dy), jnp.bfloat16),
            pltpu.VMEM((N_LAYERS, 2, C, hb, hx), jnp.bfloat16),
            pltpu.VMEM((N_LAYERS, 2, C, hb, hx), jnp.bfloat16),
            pltpu.VMEM((N_LAYERS, dk, hx), jnp.bfloat16),
            pltpu.VMEM((N_LAYERS, dk, hx), jnp.bfloat16),
            pltpu.VMEM((N_LAYERS, hx, dy), jnp.bfloat16),
            pltpu.VMEM((N_LAYERS, hx, dy), jnp.bfloat16),
            pltpu.SemaphoreType.DMA((C,)),
            pltpu.SemaphoreType.DMA((C,)),
            pltpu.SemaphoreType.DMA((C,)),
            pltpu.SemaphoreType.DMA((C,)),
            pltpu.SemaphoreType.DMA((N_LAYERS, 2, C)),
            pltpu.SemaphoreType.DMA((N_LAYERS, 2, C)),
            pltpu.SemaphoreType.DMA((N_LAYERS,)),
            pltpu.SemaphoreType.DMA((N_LAYERS,)),
            pltpu.SemaphoreType.DMA((N_LAYERS,)),
            pltpu.SemaphoreType.DMA((N_LAYERS,)),
        ],
        compiler_params=pltpu.CompilerParams(collective_id=0),
    )(x, Win0, Wout0, Win1, Wout1, Win2, Wout2)
